# Initial kernel scaffold; baseline (speedup 1.0000x reference)
import jax
import jax.numpy as jnp
from jax.experimental import pallas as pl


def kernel(user_emb, item_emb, edge_index):
    raise NotImplementedError("write your pallas kernel here")


# speedup vs baseline: 2.7390x; 2.7390x over previous
import functools

import jax
import jax.numpy as jnp
from jax import lax
from jax.experimental import pallas as pl
from jax.experimental.pallas import tpu as pltpu, tpu_sc as plsc

NC = 2
NS = 16
C = 80
WPT = 3136
ZR = 224
ZCH = 7


def _make_deg_kernel(E, NPAD):
    EPT = E // NS
    n_chunks = EPT // C

    @functools.partial(
        pl.kernel,
        out_type=jax.ShapeDtypeStruct((NC * NPAD,), jnp.float32),
        mesh=plsc.VectorSubcoreMesh(core_axis_name="c", subcore_axis_name="s"),
        compiler_params=pltpu.CompilerParams(use_tc_tiling_on_sc=False),
        scratch_types=[
            pltpu.VMEM((C,), jnp.int32),
            pltpu.VMEM((C,), jnp.float32),
            pltpu.VMEM((WPT,), jnp.float32),
            pltpu.VMEM_SHARED((NPAD,), jnp.float32),
        ],
    )
    def deg_kernel(edge_hbm, zeros_hbm, deg_hbm, idx_v, ones_v, stage_v, acc_sh):
        c = lax.axis_index("c")
        s = lax.axis_index("s")
        w0 = pl.multiple_of(s * WPT, 8)
        pltpu.sync_copy(zeros_hbm, stage_v)
        pltpu.sync_copy(stage_v, acc_sh.at[pl.ds(w0, WPT)])
        for j in range(C // 16):
            ones_v[pl.ds(16 * j, 16)] = jnp.ones((16,), jnp.float32)
        plsc.subcore_barrier()

        e0 = c * E + s * EPT

        def chunk(i, carry):
            b = pl.multiple_of(e0 + i * C, 8)
            pltpu.sync_copy(edge_hbm.at[pl.ds(b, C)], idx_v)
            pltpu.sync_copy(ones_v, acc_sh.at[idx_v], add=True)
            return carry

        lax.fori_loop(0, n_chunks, chunk, 0)
        plsc.subcore_barrier()
        o0 = pl.multiple_of(c * NPAD + w0, 8)
        pltpu.sync_copy(acc_sh.at[pl.ds(w0, WPT)], stage_v)
        pltpu.sync_copy(stage_v, deg_hbm.at[pl.ds(o0, WPT)])

    return deg_kernel


def _make_layer_kernel(N, E, D, ROWS):
    NHALF = N // NC
    EPT = E // NS
    n_chunks = EPT // C
    RPT = ROWS // NS

    @functools.partial(
        pl.kernel,
        out_type=jax.ShapeDtypeStruct((NC * ROWS, D), jnp.float32),
        mesh=plsc.VectorSubcoreMesh(core_axis_name="c", subcore_axis_name="s"),
        compiler_params=pltpu.CompilerParams(use_tc_tiling_on_sc=False),
        scratch_types=[
            pltpu.VMEM((C,), jnp.int32),
            pltpu.VMEM((C,), jnp.int32),
            pltpu.VMEM((C,), jnp.int32),
            pltpu.VMEM((C, D), jnp.float32),
            pltpu.VMEM((ZR, D), jnp.float32),
            pltpu.VMEM_SHARED((ROWS, D), jnp.float32),
            pltpu.SemaphoreType.DMA,
        ],
    )
    def layer_kernel(g_hbm, src_hbm, dst_hbm, zeros_hbm, out_hbm,
                     src_v, dst_v, dloc_v, rows_v, stage_v, acc_sh, sem):
        c = lax.axis_index("c")
        s = lax.axis_index("s")
        base_node = c * NHALF
        r0 = s * RPT
        pltpu.sync_copy(zeros_hbm, stage_v)
        for k in range(ZCH):
            pltpu.sync_copy(stage_v, acc_sh.at[pl.ds(r0 + k * ZR, ZR)])
        plsc.subcore_barrier()

        e0 = s * EPT

        def chunk(i, carry):
            b = pl.multiple_of(e0 + i * C, 8)
            pltpu.sync_copy(src_hbm.at[pl.ds(b, C)], src_v)
            pltpu.sync_copy(dst_hbm.at[pl.ds(b, C)], dst_v)
            pltpu.async_copy(g_hbm.at[src_v], rows_v, sem).wait()
            for j in range(C // 16):
                d = dst_v[pl.ds(16 * j, 16)]
                dl = d - base_node
                ok = (dl >= 0) & (dl < NHALF)
                dloc_v[pl.ds(16 * j, 16)] = jnp.where(ok, dl, NHALF)
            pltpu.sync_copy(rows_v, acc_sh.at[dloc_v], add=True)
            return carry

        lax.fori_loop(0, n_chunks, chunk, 0)
        plsc.subcore_barrier()
        o0 = c * ROWS + r0
        for k in range(ZCH):
            pltpu.sync_copy(acc_sh.at[pl.ds(r0 + k * ZR, ZR)], stage_v)
            pltpu.sync_copy(
                stage_v, out_hbm.at[pl.ds(pl.multiple_of(o0 + k * ZR, 8), ZR)])

    return layer_kernel


def _prep_body(h_ref, od_ref, id_ref, g0_ref, inorm_ref, ion_ref):
    on = lax.rsqrt(jnp.maximum(od_ref[...], 1.0))
    inn = lax.rsqrt(jnp.maximum(id_ref[...], 1.0))
    g0_ref[...] = h_ref[...] * on
    inorm_ref[...] = inn
    ion_ref[...] = inn * on


def _epi_body(agg_ref, inorm_ref, ion_ref, s_ref, g_ref, snew_ref):
    a = agg_ref[...]
    g_ref[...] = a * ion_ref[...]
    snew_ref[...] = s_ref[...] + a * inorm_ref[...]


def _fin_body(agg_ref, inorm_ref, s_ref, o_ref):
    o_ref[...] = (s_ref[...] + agg_ref[...] * inorm_ref[...]) * 0.25


def kernel(user_emb, item_emb, edge_index):
    N = user_emb.shape[0] + item_emb.shape[0]
    D = user_emb.shape[1]
    E = edge_index.shape[1]
    NHALF = N // NC
    ROWS = NS * ZCH * ZR
    NPAD = NS * WPT

    src = edge_index[0]
    dst = edge_index[1]
    h0 = jnp.concatenate([user_emb, item_emb], axis=0)

    deg_zeros = jnp.zeros((WPT,), jnp.float32)
    degs = _make_deg_kernel(E, NPAD)(edge_index.reshape(-1), deg_zeros)
    od = degs[:N, None]
    idg = degs[NPAD:NPAD + N, None]

    R = 2000
    grid = (N // R,)
    mat = pl.BlockSpec((R, D), lambda i: (i, 0))
    col = pl.BlockSpec((R, 1), lambda i: (i, 0))
    g0, inorm, ion = pl.pallas_call(
        _prep_body,
        grid=grid,
        in_specs=[mat, col, col],
        out_specs=[mat, col, col],
        out_shape=[
            jax.ShapeDtypeStruct((N, D), jnp.float32),
            jax.ShapeDtypeStruct((N, 1), jnp.float32),
            jax.ShapeDtypeStruct((N, 1), jnp.float32),
        ],
    )(h0, od, idg)

    layer = _make_layer_kernel(N, E, D, ROWS)
    layer_zeros = jnp.zeros((ZR, D), jnp.float32)

    epi = pl.pallas_call(
        _epi_body,
        grid=grid,
        in_specs=[mat, col, col, mat],
        out_specs=[mat, mat],
        out_shape=[
            jax.ShapeDtypeStruct((N, D), jnp.float32),
            jax.ShapeDtypeStruct((N, D), jnp.float32),
        ],
    )
    fin = pl.pallas_call(
        _fin_body,
        grid=grid,
        in_specs=[mat, col, mat],
        out_specs=mat,
        out_shape=jax.ShapeDtypeStruct((N, D), jnp.float32),
    )

    g = g0
    s_acc = h0
    for k in range(3):
        aggp = layer(g, src, dst, layer_zeros)
        agg = jnp.concatenate(
            [aggp[:NHALF], aggp[ROWS:ROWS + NHALF]], axis=0)
        if k < 2:
            g, s_acc = epi(agg, inorm, ion, s_acc)
        else:
            out = fin(agg, inorm, s_acc)

    return (out[: user_emb.shape[0]], out[user_emb.shape[0]:])


# speedup vs baseline: 4.9609x; 1.8112x over previous
import functools

import jax
import jax.numpy as jnp
from jax import lax
from jax.experimental import pallas as pl
from jax.experimental.pallas import tpu as pltpu, tpu_sc as plsc

NC = 2
NS = 16
C = 80
WPT = 3136
ZR = 224
ZCH = 7


def _make_deg_kernel(E, NPAD):
    EPT = E // NS
    n_chunks = EPT // C

    @functools.partial(
        pl.kernel,
        out_type=jax.ShapeDtypeStruct((NC * NPAD,), jnp.float32),
        mesh=plsc.VectorSubcoreMesh(core_axis_name="c", subcore_axis_name="s"),
        compiler_params=pltpu.CompilerParams(use_tc_tiling_on_sc=False),
        scratch_types=[
            pltpu.VMEM((C,), jnp.int32),
            pltpu.VMEM((C,), jnp.float32),
            pltpu.VMEM((WPT,), jnp.float32),
            pltpu.VMEM_SHARED((NPAD,), jnp.float32),
        ],
    )
    def deg_kernel(edge_hbm, zeros_hbm, deg_hbm, idx_v, ones_v, stage_v, acc_sh):
        c = lax.axis_index("c")
        s = lax.axis_index("s")
        w0 = pl.multiple_of(s * WPT, 8)
        pltpu.sync_copy(zeros_hbm, stage_v)
        pltpu.sync_copy(stage_v, acc_sh.at[pl.ds(w0, WPT)])
        for j in range(C // 16):
            ones_v[pl.ds(16 * j, 16)] = jnp.ones((16,), jnp.float32)
        plsc.subcore_barrier()

        e0 = c * E + s * EPT

        def chunk(i, carry):
            b = pl.multiple_of(e0 + i * C, 8)
            pltpu.sync_copy(edge_hbm.at[pl.ds(b, C)], idx_v)
            pltpu.sync_copy(ones_v, acc_sh.at[idx_v], add=True)
            return carry

        lax.fori_loop(0, n_chunks, chunk, 0)
        plsc.subcore_barrier()
        o0 = pl.multiple_of(c * NPAD + w0, 8)
        pltpu.sync_copy(acc_sh.at[pl.ds(w0, WPT)], stage_v)
        pltpu.sync_copy(stage_v, deg_hbm.at[pl.ds(o0, WPT)])

    return deg_kernel


def _make_layer_kernel(N, E, D, ROWS):
    NHALF = N // NC
    EPT = E // NS
    n_chunks = EPT // C
    RPT = ROWS // NS

    @functools.partial(
        pl.kernel,
        out_type=jax.ShapeDtypeStruct((NC * ROWS, D), jnp.float32),
        mesh=plsc.VectorSubcoreMesh(core_axis_name="c", subcore_axis_name="s"),
        compiler_params=pltpu.CompilerParams(use_tc_tiling_on_sc=False),
        scratch_types=[
            pltpu.VMEM((C,), jnp.int32),
            pltpu.VMEM((C,), jnp.int32),
            pltpu.VMEM((C,), jnp.int32),
            pltpu.VMEM((C,), jnp.int32),
            pltpu.VMEM((C,), jnp.int32),
            pltpu.VMEM((C,), jnp.int32),
            pltpu.VMEM((C, D), jnp.float32),
            pltpu.VMEM((C, D), jnp.float32),
            pltpu.VMEM((ZR, D), jnp.float32),
            pltpu.VMEM_SHARED((ROWS, D), jnp.float32),
            pltpu.SemaphoreType.DMA,
            pltpu.SemaphoreType.DMA,
            pltpu.SemaphoreType.DMA,
        ],
    )
    def layer_kernel(g_hbm, src_hbm, dst_hbm, zeros_hbm, out_hbm,
                     src_a, src_b, dst_a, dst_b, dloc_a, dloc_b,
                     rows_a, rows_b, stage_v, acc_sh, sem_i, sem_g, sem_s):
        c = lax.axis_index("c")
        s = lax.axis_index("s")
        base_node = c * NHALF
        r0 = s * RPT
        pltpu.sync_copy(zeros_hbm, stage_v)
        for k in range(ZCH):
            pltpu.sync_copy(stage_v, acc_sh.at[pl.ds(r0 + k * ZR, ZR)])
        plsc.subcore_barrier()

        e0 = s * EPT
        srcs = (src_a, src_b)
        dsts = (dst_a, dst_b)
        dlocs = (dloc_a, dloc_b)
        rows = (rows_a, rows_b)

        def remap(dst_v, dloc_v):
            for j in range(C // 16):
                d = dst_v[pl.ds(16 * j, 16)]
                dl = d - base_node
                ok = (dl >= 0) & (dl < NHALF)
                dloc_v[pl.ds(16 * j, 16)] = jnp.where(ok, dl, NHALF)

        for j in range(C // 16):
            dloc_a[pl.ds(16 * j, 16)] = jnp.zeros((16,), jnp.int32)
        pltpu.async_copy(stage_v.at[pl.ds(0, C)], acc_sh.at[dloc_a], sem_s,
                         add=True)
        b0 = pl.multiple_of(e0, 8)
        pltpu.sync_copy(src_hbm.at[pl.ds(b0, C)], src_a)
        pltpu.sync_copy(dst_hbm.at[pl.ds(b0, C)], dst_a)
        pltpu.async_copy(g_hbm.at[src_a], rows_a, sem_g)

        def phase(e, t, n):
            bn = pl.multiple_of(e0 + (e + 1) * C, 8)
            pltpu.async_copy(src_hbm.at[pl.ds(bn, C)], srcs[n], sem_i)
            pltpu.async_copy(dst_hbm.at[pl.ds(bn, C)], dsts[n], sem_i)
            pltpu.make_async_copy(g_hbm.at[srcs[t]], rows[t], sem_g).wait()
            pltpu.make_async_copy(
                rows[n], acc_sh.at[dlocs[n]], sem_s).wait()
            pltpu.make_async_copy(src_hbm.at[pl.ds(bn, C)], srcs[n],
                                  sem_i).wait()
            pltpu.make_async_copy(dst_hbm.at[pl.ds(bn, C)], dsts[n],
                                  sem_i).wait()
            pltpu.async_copy(g_hbm.at[srcs[n]], rows[n], sem_g)
            remap(dsts[t], dlocs[t])
            pltpu.async_copy(rows[t], acc_sh.at[dlocs[t]], sem_s, add=True)

        def pair(p, carry):
            phase(2 * p, 0, 1)
            phase(2 * p + 1, 1, 0)
            return carry

        lax.fori_loop(0, (n_chunks - 1) // 2, pair, 0)
        pltpu.make_async_copy(g_hbm.at[src_a], rows_a, sem_g).wait()
        pltpu.make_async_copy(rows_b, acc_sh.at[dloc_b], sem_s).wait()
        remap(dst_a, dloc_a)
        pltpu.async_copy(rows_a, acc_sh.at[dloc_a], sem_s, add=True)
        pltpu.make_async_copy(rows_a, acc_sh.at[dloc_a], sem_s).wait()
        plsc.subcore_barrier()
        o0 = c * ROWS + r0
        for k in range(ZCH):
            pltpu.sync_copy(acc_sh.at[pl.ds(r0 + k * ZR, ZR)], stage_v)
            pltpu.sync_copy(
                stage_v, out_hbm.at[pl.ds(pl.multiple_of(o0 + k * ZR, 8), ZR)])

    return layer_kernel


def _prep_body(h_ref, od_ref, id_ref, g0_ref, inorm_ref, ion_ref):
    on = lax.rsqrt(jnp.maximum(od_ref[...], 1.0))
    inn = lax.rsqrt(jnp.maximum(id_ref[...], 1.0))
    g0_ref[...] = h_ref[...] * on
    inorm_ref[...] = inn
    ion_ref[...] = inn * on


def _epi_body(agg_ref, inorm_ref, ion_ref, s_ref, g_ref, snew_ref):
    a = agg_ref[...]
    g_ref[...] = a * ion_ref[...]
    snew_ref[...] = s_ref[...] + a * inorm_ref[...]


def _fin_body(agg_ref, inorm_ref, s_ref, o_ref):
    o_ref[...] = (s_ref[...] + agg_ref[...] * inorm_ref[...]) * 0.25


def kernel(user_emb, item_emb, edge_index):
    N = user_emb.shape[0] + item_emb.shape[0]
    D = user_emb.shape[1]
    E = edge_index.shape[1]
    NHALF = N // NC
    ROWS = NS * ZCH * ZR
    NPAD = NS * WPT

    src = edge_index[0]
    dst = edge_index[1]
    h0 = jnp.concatenate([user_emb, item_emb], axis=0)

    deg_zeros = jnp.zeros((WPT,), jnp.float32)
    degs = _make_deg_kernel(E, NPAD)(edge_index.reshape(-1), deg_zeros)
    od = degs[:N, None]
    idg = degs[NPAD:NPAD + N, None]

    R = 2000
    grid = (N // R,)
    mat = pl.BlockSpec((R, D), lambda i: (i, 0))
    col = pl.BlockSpec((R, 1), lambda i: (i, 0))
    g0, inorm, ion = pl.pallas_call(
        _prep_body,
        grid=grid,
        in_specs=[mat, col, col],
        out_specs=[mat, col, col],
        out_shape=[
            jax.ShapeDtypeStruct((N, D), jnp.float32),
            jax.ShapeDtypeStruct((N, 1), jnp.float32),
            jax.ShapeDtypeStruct((N, 1), jnp.float32),
        ],
    )(h0, od, idg)

    layer = _make_layer_kernel(N, E, D, ROWS)
    layer_zeros = jnp.zeros((ZR, D), jnp.float32)

    epi = pl.pallas_call(
        _epi_body,
        grid=grid,
        in_specs=[mat, col, col, mat],
        out_specs=[mat, mat],
        out_shape=[
            jax.ShapeDtypeStruct((N, D), jnp.float32),
            jax.ShapeDtypeStruct((N, D), jnp.float32),
        ],
    )
    fin = pl.pallas_call(
        _fin_body,
        grid=grid,
        in_specs=[mat, col, mat],
        out_specs=mat,
        out_shape=jax.ShapeDtypeStruct((N, D), jnp.float32),
    )

    g = g0
    s_acc = h0
    for k in range(3):
        aggp = layer(g, src, dst, layer_zeros)
        agg = jnp.concatenate(
            [aggp[:NHALF], aggp[ROWS:ROWS + NHALF]], axis=0)
        if k < 2:
            g, s_acc = epi(agg, inorm, ion, s_acc)
        else:
            out = fin(agg, inorm, s_acc)

    return (out[: user_emb.shape[0]], out[user_emb.shape[0]:])


# speedup vs baseline: 5.3260x; 1.0736x over previous
import functools

import jax
import jax.numpy as jnp
from jax import lax
from jax.experimental import pallas as pl
from jax.experimental.pallas import tpu as pltpu, tpu_sc as plsc

NC = 2
NS = 16
C = 80
WPT = 3136
ZR = 112
ZCH = 14


def _make_deg_kernel(E, NPAD):
    EPT = E // NS
    n_chunks = EPT // C

    assert n_chunks % 2 == 1 and EPT % C == 0

    @functools.partial(
        pl.kernel,
        out_type=jax.ShapeDtypeStruct((NC * NPAD,), jnp.float32),
        mesh=plsc.VectorSubcoreMesh(core_axis_name="c", subcore_axis_name="s"),
        compiler_params=pltpu.CompilerParams(use_tc_tiling_on_sc=False),
        scratch_types=[
            pltpu.VMEM((C,), jnp.int32),
            pltpu.VMEM((C,), jnp.int32),
            pltpu.VMEM((C,), jnp.float32),
            pltpu.VMEM((C,), jnp.float32),
            pltpu.VMEM((WPT,), jnp.float32),
            pltpu.VMEM_SHARED((NPAD,), jnp.float32),
            pltpu.SemaphoreType.DMA,
            pltpu.SemaphoreType.DMA,
        ],
    )
    def deg_kernel(edge_hbm, zeros_hbm, deg_hbm, idx_a, idx_b, ones_v,
                   zeros_v, stage_v, acc_sh, sem_i, sem_s):
        c = lax.axis_index("c")
        s = lax.axis_index("s")
        w0 = pl.multiple_of(s * WPT, 8)
        pltpu.sync_copy(zeros_hbm, stage_v)
        pltpu.sync_copy(stage_v, acc_sh.at[pl.ds(w0, WPT)])
        for j in range(C // 16):
            ones_v[pl.ds(16 * j, 16)] = jnp.ones((16,), jnp.float32)
            zeros_v[pl.ds(16 * j, 16)] = jnp.zeros((16,), jnp.float32)
            idx_b[pl.ds(16 * j, 16)] = jnp.zeros((16,), jnp.int32)
        plsc.subcore_barrier()

        e0 = c * E + s * EPT
        idxs = (idx_a, idx_b)
        pltpu.async_copy(zeros_v, acc_sh.at[idx_b], sem_s, add=True)
        pltpu.async_copy(edge_hbm.at[pl.ds(pl.multiple_of(e0, 8), C)], idx_a,
                         sem_i)

        def phase(e, t, n, issue_next):
            pltpu.make_async_copy(
                ones_v, acc_sh.at[idxs[n]], sem_s).wait()
            if issue_next:
                bn = pl.multiple_of(e0 + (e + 1) * C, 8)
                pltpu.async_copy(edge_hbm.at[pl.ds(bn, C)], idxs[n], sem_i)
            b = pl.multiple_of(e0 + e * C, 8)
            pltpu.make_async_copy(edge_hbm.at[pl.ds(b, C)], idxs[t],
                                  sem_i).wait()
            pltpu.async_copy(ones_v, acc_sh.at[idxs[t]], sem_s, add=True)

        def pair(p, carry):
            phase(2 * p, 0, 1, True)
            phase(2 * p + 1, 1, 0, True)
            return carry

        lax.fori_loop(0, (n_chunks - 1) // 2, pair, 0)
        phase(n_chunks - 1, 0, 1, False)
        pltpu.make_async_copy(ones_v, acc_sh.at[idx_a], sem_s).wait()
        plsc.subcore_barrier()
        o0 = pl.multiple_of(c * NPAD + w0, 8)
        pltpu.sync_copy(acc_sh.at[pl.ds(w0, WPT)], stage_v)
        pltpu.sync_copy(stage_v, deg_hbm.at[pl.ds(o0, WPT)])

    return deg_kernel


def _make_layer_kernel(N, E, D, ROWS):
    NHALF = N // NC
    EPT = E // NS
    CL = 128
    n_full = EPT // CL
    TAIL = EPT - n_full * CL
    RPT = ROWS // NS
    assert n_full % 2 == 0 and n_full >= 4 and TAIL % 8 == 0 and 0 < TAIL <= 128

    @functools.partial(
        pl.kernel,
        out_type=jax.ShapeDtypeStruct((NC * ROWS, D), jnp.float32),
        mesh=plsc.VectorSubcoreMesh(core_axis_name="c", subcore_axis_name="s"),
        compiler_params=pltpu.CompilerParams(use_tc_tiling_on_sc=False),
        scratch_types=[
            pltpu.VMEM((CL,), jnp.int32),
            pltpu.VMEM((CL,), jnp.int32),
            pltpu.VMEM((CL,), jnp.int32),
            pltpu.VMEM((CL,), jnp.int32),
            pltpu.VMEM((CL,), jnp.int32),
            pltpu.VMEM((CL,), jnp.int32),
            pltpu.VMEM((CL, D), jnp.float32),
            pltpu.VMEM((CL, D), jnp.float32),
            pltpu.VMEM((TAIL,), jnp.int32),
            pltpu.VMEM((TAIL,), jnp.int32),
            pltpu.VMEM((TAIL,), jnp.int32),
            pltpu.VMEM((TAIL, D), jnp.float32),
            pltpu.VMEM((ZR, D), jnp.float32),
            pltpu.VMEM_SHARED((ROWS, D), jnp.float32),
            pltpu.SemaphoreType.DMA,
            pltpu.SemaphoreType.DMA,
            pltpu.SemaphoreType.DMA,
        ],
    )
    def layer_kernel(g_hbm, src_hbm, dst_hbm, zeros_hbm, out_hbm,
                     src_a, src_b, dst_a, dst_b, dloc_a, dloc_b,
                     rows_a, rows_b, src_t, dst_t, dloc_t, rows_t,
                     stage_v, acc_sh, sem_i, sem_g, sem_s):
        c = lax.axis_index("c")
        s = lax.axis_index("s")
        base_node = c * NHALF
        r0 = s * RPT
        pltpu.sync_copy(zeros_hbm.at[pl.ds(0, ZR)], stage_v)
        for k in range(ZCH):
            pltpu.sync_copy(stage_v, acc_sh.at[pl.ds(r0 + k * ZR, ZR)])
        plsc.subcore_barrier()

        e0 = s * EPT
        srcs = (src_a, src_b)
        dsts = (dst_a, dst_b)
        dlocs = (dloc_a, dloc_b)
        rows = (rows_a, rows_b)

        def remap(dst_v, dloc_v, n16):
            for j in range(n16):
                d = dst_v[pl.ds(16 * j, 16)]
                dl = d - base_node
                ok = (dl >= 0) & (dl < NHALF)
                dloc_v[pl.ds(16 * j, 16)] = jnp.where(ok, dl, NHALF)

        for j in range(CL // 16):
            dloc_a[pl.ds(16 * j, 16)] = jnp.zeros((16,), jnp.int32)
        pltpu.sync_copy(zeros_hbm, rows_b)
        pltpu.async_copy(rows_b, acc_sh.at[dloc_a], sem_s, add=True)
        b0 = pl.multiple_of(e0, 8)
        pltpu.sync_copy(src_hbm.at[pl.ds(b0, CL)], src_a)
        pltpu.sync_copy(dst_hbm.at[pl.ds(b0, CL)], dst_a)
        pltpu.async_copy(g_hbm.at[src_a], rows_a, sem_g)

        def phase(e, t, n, issue_next):
            if issue_next:
                bn = pl.multiple_of(e0 + (e + 1) * CL, 8)
                pltpu.async_copy(src_hbm.at[pl.ds(bn, CL)], srcs[n], sem_i)
                pltpu.async_copy(dst_hbm.at[pl.ds(bn, CL)], dsts[n], sem_i)
            pltpu.make_async_copy(g_hbm.at[srcs[t]], rows[t], sem_g).wait()
            pltpu.make_async_copy(
                rows[n], acc_sh.at[dlocs[n]], sem_s).wait()
            if issue_next:
                bn = pl.multiple_of(e0 + (e + 1) * CL, 8)
                pltpu.make_async_copy(src_hbm.at[pl.ds(bn, CL)], srcs[n],
                                      sem_i).wait()
                pltpu.make_async_copy(dst_hbm.at[pl.ds(bn, CL)], dsts[n],
                                      sem_i).wait()
                pltpu.async_copy(g_hbm.at[srcs[n]], rows[n], sem_g)
            remap(dsts[t], dlocs[t], CL // 16)
            pltpu.async_copy(rows[t], acc_sh.at[dlocs[t]], sem_s, add=True)

        def pair(p, carry):
            phase(2 * p, 0, 1, True)
            phase(2 * p + 1, 1, 0, True)
            return carry

        lax.fori_loop(0, (n_full - 2) // 2, pair, 0)
        phase(n_full - 2, 0, 1, True)
        phase(n_full - 1, 1, 0, False)
        bt = pl.multiple_of(e0 + n_full * CL, 8)
        pltpu.sync_copy(src_hbm.at[pl.ds(bt, TAIL)], src_t)
        pltpu.sync_copy(dst_hbm.at[pl.ds(bt, TAIL)], dst_t)
        pltpu.async_copy(g_hbm.at[src_t], rows_t, sem_g).wait()
        remap(dst_t, dloc_t, TAIL // 16)
        pltpu.async_copy(rows_t, acc_sh.at[dloc_t], sem_s, add=True)
        pltpu.make_async_copy(rows_b, acc_sh.at[dloc_b], sem_s).wait()
        pltpu.make_async_copy(rows_t, acc_sh.at[dloc_t], sem_s).wait()
        plsc.subcore_barrier()
        o0 = c * ROWS + r0
        for k in range(ZCH):
            pltpu.sync_copy(acc_sh.at[pl.ds(r0 + k * ZR, ZR)], stage_v)
            pltpu.sync_copy(
                stage_v, out_hbm.at[pl.ds(pl.multiple_of(o0 + k * ZR, 8), ZR)])

    return layer_kernel


def _prep_body(h_ref, od_ref, id_ref, g0_ref, inorm_ref, ion_ref):
    on = lax.rsqrt(jnp.maximum(od_ref[...], 1.0))
    inn = lax.rsqrt(jnp.maximum(id_ref[...], 1.0))
    g0_ref[...] = h_ref[...] * on
    inorm_ref[...] = inn
    ion_ref[...] = inn * on


def _epi_body(agg_ref, inorm_ref, ion_ref, s_ref, g_ref, snew_ref):
    a = agg_ref[...]
    g_ref[...] = a * ion_ref[...]
    snew_ref[...] = s_ref[...] + a * inorm_ref[...]


def _fin_body(agg_ref, inorm_ref, s_ref, o_ref):
    o_ref[...] = (s_ref[...] + agg_ref[...] * inorm_ref[...]) * 0.25


def kernel(user_emb, item_emb, edge_index):
    N = user_emb.shape[0] + item_emb.shape[0]
    D = user_emb.shape[1]
    E = edge_index.shape[1]
    NHALF = N // NC
    ROWS = NS * ZCH * ZR
    NPAD = NS * WPT

    src = edge_index[0]
    dst = edge_index[1]
    h0 = jnp.concatenate([user_emb, item_emb], axis=0)

    deg_zeros = jnp.zeros((WPT,), jnp.float32)
    degs = _make_deg_kernel(E, NPAD)(edge_index.reshape(-1), deg_zeros)
    od = degs[:N, None]
    idg = degs[NPAD:NPAD + N, None]

    R = 2000
    grid = (N // R,)
    mat = pl.BlockSpec((R, D), lambda i: (i, 0))
    col = pl.BlockSpec((R, 1), lambda i: (i, 0))
    g0, inorm, ion = pl.pallas_call(
        _prep_body,
        grid=grid,
        in_specs=[mat, col, col],
        out_specs=[mat, col, col],
        out_shape=[
            jax.ShapeDtypeStruct((N, D), jnp.float32),
            jax.ShapeDtypeStruct((N, 1), jnp.float32),
            jax.ShapeDtypeStruct((N, 1), jnp.float32),
        ],
    )(h0, od, idg)

    layer = _make_layer_kernel(N, E, D, ROWS)
    layer_zeros = jnp.zeros((128, D), jnp.float32)

    epi = pl.pallas_call(
        _epi_body,
        grid=grid,
        in_specs=[mat, col, col, mat],
        out_specs=[mat, mat],
        out_shape=[
            jax.ShapeDtypeStruct((N, D), jnp.float32),
            jax.ShapeDtypeStruct((N, D), jnp.float32),
        ],
    )
    fin = pl.pallas_call(
        _fin_body,
        grid=grid,
        in_specs=[mat, col, mat],
        out_specs=mat,
        out_shape=jax.ShapeDtypeStruct((N, D), jnp.float32),
    )

    g = g0
    s_acc = h0
    for k in range(3):
        aggp = layer(g, src, dst, layer_zeros)
        agg = jnp.concatenate(
            [aggp[:NHALF], aggp[ROWS:ROWS + NHALF]], axis=0)
        if k < 2:
            g, s_acc = epi(agg, inorm, ion, s_acc)
        else:
            out = fin(agg, inorm, s_acc)

    return (out[: user_emb.shape[0]], out[user_emb.shape[0]:])


# speedup vs baseline: 7.5772x; 1.4227x over previous
import functools

import jax
import jax.numpy as jnp
from jax import lax
from jax.experimental import pallas as pl
from jax.experimental.pallas import tpu as pltpu, tpu_sc as plsc

NC = 2
NS = 16
C = 80
WPT = 3136
ZR = 112
ZCH = 14


def _make_deg_kernel(E, NPAD):
    EPT = E // NS
    n_chunks = EPT // C

    assert n_chunks % 2 == 1 and EPT % C == 0

    @functools.partial(
        pl.kernel,
        out_type=jax.ShapeDtypeStruct((NC * NPAD,), jnp.float32),
        mesh=plsc.VectorSubcoreMesh(core_axis_name="c", subcore_axis_name="s"),
        compiler_params=pltpu.CompilerParams(use_tc_tiling_on_sc=False, needs_layout_passes=False),
        scratch_types=[
            pltpu.VMEM((C,), jnp.int32),
            pltpu.VMEM((C,), jnp.int32),
            pltpu.VMEM((C,), jnp.float32),
            pltpu.VMEM((C,), jnp.float32),
            pltpu.VMEM((WPT,), jnp.float32),
            pltpu.VMEM_SHARED((NPAD,), jnp.float32),
            pltpu.SemaphoreType.DMA,
            pltpu.SemaphoreType.DMA,
        ],
    )
    def deg_kernel(edge_hbm, zeros_hbm, deg_hbm, idx_a, idx_b, ones_v,
                   zeros_v, stage_v, acc_sh, sem_i, sem_s):
        c = lax.axis_index("c")
        s = lax.axis_index("s")
        w0 = pl.multiple_of(s * WPT, 8)
        pltpu.sync_copy(zeros_hbm, stage_v)
        pltpu.sync_copy(stage_v, acc_sh.at[pl.ds(w0, WPT)])
        for j in range(C // 16):
            ones_v[pl.ds(16 * j, 16)] = jnp.ones((16,), jnp.float32)
            zeros_v[pl.ds(16 * j, 16)] = jnp.zeros((16,), jnp.float32)
            idx_b[pl.ds(16 * j, 16)] = jnp.zeros((16,), jnp.int32)
        plsc.subcore_barrier()

        e0 = c * E + s * EPT
        idxs = (idx_a, idx_b)
        pltpu.async_copy(zeros_v, acc_sh.at[idx_b], sem_s, add=True)
        pltpu.async_copy(edge_hbm.at[pl.ds(pl.multiple_of(e0, 8), C)], idx_a,
                         sem_i)

        def phase(e, t, n, issue_next):
            pltpu.make_async_copy(
                ones_v, acc_sh.at[idxs[n]], sem_s).wait()
            if issue_next:
                bn = pl.multiple_of(e0 + (e + 1) * C, 8)
                pltpu.async_copy(edge_hbm.at[pl.ds(bn, C)], idxs[n], sem_i)
            b = pl.multiple_of(e0 + e * C, 8)
            pltpu.make_async_copy(edge_hbm.at[pl.ds(b, C)], idxs[t],
                                  sem_i).wait()
            pltpu.async_copy(ones_v, acc_sh.at[idxs[t]], sem_s, add=True)

        def pair(p, carry):
            phase(2 * p, 0, 1, True)
            phase(2 * p + 1, 1, 0, True)
            return carry

        lax.fori_loop(0, (n_chunks - 1) // 2, pair, 0)
        phase(n_chunks - 1, 0, 1, False)
        pltpu.make_async_copy(ones_v, acc_sh.at[idx_a], sem_s).wait()
        plsc.subcore_barrier()
        o0 = pl.multiple_of(c * NPAD + w0, 8)
        pltpu.sync_copy(acc_sh.at[pl.ds(w0, WPT)], stage_v)
        pltpu.sync_copy(stage_v, deg_hbm.at[pl.ds(o0, WPT)])

    return deg_kernel


def _make_part_kernel(N, E, RS):
    NHALF = N // NC
    EPT = E // NS
    CL = 128
    n_full = EPT // CL
    TAIL = EPT - n_full * CL
    assert n_full % 2 == 0 and TAIL % 16 == 0 and 0 < TAIL <= CL

    @functools.partial(
        pl.kernel,
        out_type=[
            jax.ShapeDtypeStruct((NC * NS * RS,), jnp.int32),
            jax.ShapeDtypeStruct((NC * NS * RS,), jnp.int32),
            jax.ShapeDtypeStruct((NC * NS,), jnp.int32),
            jax.ShapeDtypeStruct((512,), jnp.int32),
        ],
        mesh=plsc.VectorSubcoreMesh(core_axis_name="c", subcore_axis_name="s"),
        compiler_params=pltpu.CompilerParams(use_tc_tiling_on_sc=False, needs_layout_passes=False),
        scratch_types=[
            pltpu.VMEM((CL,), jnp.int32),
            pltpu.VMEM((CL,), jnp.int32),
            pltpu.VMEM((CL,), jnp.int32),
            pltpu.VMEM((CL,), jnp.int32),
            pltpu.VMEM((256,), jnp.int32),
            pltpu.VMEM((256,), jnp.int32),
            pltpu.VMEM((CL,), jnp.int32),
            pltpu.VMEM((CL,), jnp.int32),
            pltpu.VMEM((16,), jnp.int32),
            pltpu.VMEM((16,), jnp.int32),
            pltpu.VMEM_SHARED((16,), jnp.int32),
            pltpu.SemaphoreType.DMA,
            pltpu.SemaphoreType.DMA,
            pltpu.SemaphoreType.DMA,
        ],
    )
    def part_kernel(src_hbm, dst_hbm, psrc_hbm, pdloc_hbm, counts_hbm,
                    waste_hbm, src_a, src_b, dst_a, dst_b, ring_s, ring_d,
                    dum_s, dum_d, cbuf, idxc, counts_sh, sem_i, sem_f, sem_c):
        c = lax.axis_index("c")
        s = lax.axis_index("s")
        base_node = c * NHALF
        region = (c * NS + s) * RS
        e0 = s * EPT
        srcs = (src_a, src_b)
        dsts = (dst_a, dst_b)
        iota = lax.iota(jnp.int32, 16)
        for j in range(CL // 16):
            dum_s[pl.ds(16 * j, 16)] = jnp.zeros((16,), jnp.int32)
            dum_d[pl.ds(16 * j, 16)] = jnp.full((16,), NHALF, jnp.int32)

        def flush_from(ref_s, ref_d, off, w):
            pltpu.make_async_copy(ring_s.at[pl.ds(0, CL)],
                                  waste_hbm.at[pl.ds(0, CL)], sem_f).wait()
            pltpu.make_async_copy(ring_d.at[pl.ds(0, CL)],
                                  waste_hbm.at[pl.ds(0, CL)], sem_f).wait()
            b = pl.multiple_of(region + w * CL, 8)
            pltpu.async_copy(ref_s.at[pl.ds(off, CL)],
                             psrc_hbm.at[pl.ds(b, CL)], sem_f)
            pltpu.async_copy(ref_d.at[pl.ds(off, CL)],
                             pdloc_hbm.at[pl.ds(b, CL)], sem_f)

        pltpu.async_copy(ring_s.at[pl.ds(0, CL)],
                         waste_hbm.at[pl.ds(0, CL)], sem_f)
        pltpu.async_copy(ring_d.at[pl.ds(0, CL)],
                         waste_hbm.at[pl.ds(128, CL)], sem_f)
        pltpu.async_copy(ring_s.at[pl.ds(128, CL)],
                         waste_hbm.at[pl.ds(256, CL)], sem_f)
        pltpu.async_copy(ring_d.at[pl.ds(128, CL)],
                         waste_hbm.at[pl.ds(384, CL)], sem_f)

        def group(sv, dv, cnt):
            dl = dv - base_node
            flag = (dl >= 0) & (dl < NHALF)
            fi = jnp.where(flag, 1, 0)
            pos = (cnt + plsc.cumsum(fi) - 1) & 255
            plsc.store_scatter(ring_s, [pos], sv, mask=flag)
            plsc.store_scatter(ring_d, [pos], dl, mask=flag)
            return cnt + jnp.sum(fi)

        def consume(t, n16, cnt0):
            w0 = cnt0 >> 7
            cnt = cnt0
            for j in range(n16):
                sv = srcs[t][pl.ds(16 * j, 16)]
                dv = dsts[t][pl.ds(16 * j, 16)]
                cnt = group(sv, dv, cnt)
            w1 = cnt >> 7

            @pl.when(w1 > w0)
            def _():
                flush_from(ring_s, ring_d,
                           pl.multiple_of((w0 & 1) * CL, 8), w0)

            return cnt

        def phase(e, t, n, issue_next, cnt):
            if issue_next:
                bn = pl.multiple_of(e0 + (e + 1) * CL, 8)
                pltpu.async_copy(src_hbm.at[pl.ds(bn, CL)], srcs[n], sem_i)
                pltpu.async_copy(dst_hbm.at[pl.ds(bn, CL)], dsts[n], sem_i)
            b = pl.multiple_of(e0 + e * CL, 8)
            pltpu.make_async_copy(src_hbm.at[pl.ds(b, CL)], srcs[t],
                                  sem_i).wait()
            pltpu.make_async_copy(dst_hbm.at[pl.ds(b, CL)], dsts[t],
                                  sem_i).wait()
            return consume(t, CL // 16, cnt)

        b0 = pl.multiple_of(e0, 8)
        pltpu.async_copy(src_hbm.at[pl.ds(b0, CL)], src_a, sem_i)
        pltpu.async_copy(dst_hbm.at[pl.ds(b0, CL)], dst_a, sem_i)

        def pair(p, cnt):
            cnt = phase(2 * p, 0, 1, True, cnt)
            cnt = phase(2 * p + 1, 1, 0, True, cnt)
            return cnt

        cnt = lax.fori_loop(0, (n_full - 2) // 2, pair, 0)
        cnt = phase(n_full - 2, 0, 1, True, cnt)
        cnt = phase(n_full - 1, 1, 0, False, cnt)
        bt = pl.multiple_of(e0 + n_full * CL, 8)
        pltpu.sync_copy(src_hbm.at[pl.ds(bt, TAIL)], src_a.at[pl.ds(0, TAIL)])
        pltpu.sync_copy(dst_hbm.at[pl.ds(bt, TAIL)], dst_a.at[pl.ds(0, TAIL)])
        cnt = consume(0, TAIL // 16, cnt)

        w = cnt >> 7
        rem = cnt & 127
        hb = pl.multiple_of((w & 1) * CL, 8)
        for j in range(CL // 16):
            lane = iota + 16 * j
            over = lane >= rem
            gs = ring_s[pl.ds(hb + 16 * j, 16)]
            gd = ring_d[pl.ds(hb + 16 * j, 16)]
            ring_s[pl.ds(hb + 16 * j, 16)] = jnp.where(over, 0, gs)
            ring_d[pl.ds(hb + 16 * j, 16)] = jnp.where(over, NHALF, gd)
        flush_from(ring_s, ring_d, hb, w)
        nw = w + 1

        @pl.when(nw % 2 == 1)
        def _():
            flush_from(dum_s, dum_d, 0, nw)

        nw2 = nw + (nw % 2)
        for _ in range(4):
            pltpu.make_async_copy(ring_s.at[pl.ds(0, CL)],
                                  waste_hbm.at[pl.ds(0, CL)], sem_f).wait()
        cbuf[pl.ds(0, 16)] = jnp.full((16,), nw2 * CL, jnp.int32)
        idxc[pl.ds(0, 16)] = jnp.full((16,), s, jnp.int32)
        pltpu.async_copy(cbuf, counts_sh.at[idxc], sem_c).wait()
        plsc.subcore_barrier()

        @pl.when(s == 0)
        def _():
            pltpu.sync_copy(counts_sh, counts_hbm.at[pl.ds(c * NS, NS)])

    return part_kernel


def _make_layer_kernel(N, E, D, ROWS, RS):
    NHALF = N // NC
    CL = 128
    RPT = ROWS // NS

    @functools.partial(
        pl.kernel,
        out_type=jax.ShapeDtypeStruct((NC * ROWS, D), jnp.float32),
        mesh=plsc.VectorSubcoreMesh(core_axis_name="c", subcore_axis_name="s"),
        compiler_params=pltpu.CompilerParams(use_tc_tiling_on_sc=False, needs_layout_passes=False),
        scratch_types=[
            pltpu.VMEM((CL,), jnp.int32),
            pltpu.VMEM((CL,), jnp.int32),
            pltpu.VMEM((CL,), jnp.int32),
            pltpu.VMEM((CL,), jnp.int32),
            pltpu.VMEM((CL, D), jnp.float32),
            pltpu.VMEM((CL, D), jnp.float32),
            pltpu.VMEM((16,), jnp.int32),
            pltpu.VMEM((ZR, D), jnp.float32),
            pltpu.VMEM_SHARED((ROWS, D), jnp.float32),
            pltpu.SemaphoreType.DMA,
            pltpu.SemaphoreType.DMA,
            pltpu.SemaphoreType.DMA,
        ],
    )
    def layer_kernel(g_hbm, psrc_hbm, pdloc_hbm, counts_hbm, zeros_hbm,
                     out_hbm, src_a, src_b, dloc_a, dloc_b,
                     rows_a, rows_b, cnt_v, stage_v, acc_sh,
                     sem_i, sem_g, sem_s):
        c = lax.axis_index("c")
        s = lax.axis_index("s")
        r0 = s * RPT
        region = (c * NS + s) * RS
        pltpu.sync_copy(zeros_hbm.at[pl.ds(0, ZR)], stage_v)
        for k in range(ZCH):
            pltpu.sync_copy(stage_v, acc_sh.at[pl.ds(r0 + k * ZR, ZR)])
        plsc.subcore_barrier()

        pltpu.sync_copy(
            counts_hbm.at[pl.ds(pl.multiple_of(c * NS, 8), NS)], cnt_v)
        cv = cnt_v[pl.ds(0, 16)]
        cw = jnp.sum(jnp.where(lax.iota(jnp.int32, 16) == s, cv, 0))
        n_t = cw >> 7

        srcs = (src_a, src_b)
        dlocs = (dloc_a, dloc_b)
        rows = (rows_a, rows_b)

        for j in range(CL // 16):
            dloc_b[pl.ds(16 * j, 16)] = jnp.zeros((16,), jnp.int32)
        pltpu.sync_copy(zeros_hbm, rows_b)
        pltpu.async_copy(rows_b, acc_sh.at[dloc_b], sem_s, add=True)
        b0 = pl.multiple_of(region, 8)
        pltpu.sync_copy(psrc_hbm.at[pl.ds(b0, CL)], src_a)
        pltpu.sync_copy(pdloc_hbm.at[pl.ds(b0, CL)], dloc_a)
        pltpu.async_copy(g_hbm.at[src_a], rows_a, sem_g)

        def phase(e, t, n, issue_next):
            pltpu.make_async_copy(
                rows[n], acc_sh.at[dlocs[n]], sem_s).wait()
            if issue_next:
                bn = pl.multiple_of(region + (e + 1) * CL, 8)
                pltpu.async_copy(psrc_hbm.at[pl.ds(bn, CL)], srcs[n], sem_i)
                pltpu.async_copy(pdloc_hbm.at[pl.ds(bn, CL)], dlocs[n], sem_i)
            pltpu.make_async_copy(g_hbm.at[srcs[t]], rows[t], sem_g).wait()
            if issue_next:
                bn = pl.multiple_of(region + (e + 1) * CL, 8)
                pltpu.make_async_copy(psrc_hbm.at[pl.ds(bn, CL)], srcs[n],
                                      sem_i).wait()
                pltpu.make_async_copy(pdloc_hbm.at[pl.ds(bn, CL)], dlocs[n],
                                      sem_i).wait()
                pltpu.async_copy(g_hbm.at[srcs[n]], rows[n], sem_g)
            pltpu.async_copy(rows[t], acc_sh.at[dlocs[t]], sem_s, add=True)

        def pair(p, carry):
            phase(2 * p, 0, 1, True)
            phase(2 * p + 1, 1, 0, True)
            return carry

        lax.fori_loop(0, (n_t - 2) // 2, pair, 0)
        phase(n_t - 2, 0, 1, True)
        phase(n_t - 1, 1, 0, False)
        pltpu.make_async_copy(rows_b, acc_sh.at[dloc_b], sem_s).wait()
        plsc.subcore_barrier()
        o0 = c * ROWS + r0
        for k in range(ZCH):
            pltpu.sync_copy(acc_sh.at[pl.ds(r0 + k * ZR, ZR)], stage_v)
            pltpu.sync_copy(
                stage_v, out_hbm.at[pl.ds(pl.multiple_of(o0 + k * ZR, 8), ZR)])

    return layer_kernel


def _prep_body(h_ref, od_ref, id_ref, g0_ref, inorm_ref, ion_ref):
    on = lax.rsqrt(jnp.maximum(od_ref[...], 1.0))
    inn = lax.rsqrt(jnp.maximum(id_ref[...], 1.0))
    g0_ref[...] = h_ref[...] * on
    inorm_ref[...] = inn
    ion_ref[...] = inn * on


def _epi_body(agg_ref, inorm_ref, ion_ref, s_ref, g_ref, snew_ref):
    a = agg_ref[...]
    g_ref[...] = a * ion_ref[...]
    snew_ref[...] = s_ref[...] + a * inorm_ref[...]


def _fin_body(agg_ref, inorm_ref, s_ref, o_ref):
    o_ref[...] = (s_ref[...] + agg_ref[...] * inorm_ref[...]) * 0.25


def kernel(user_emb, item_emb, edge_index):
    N = user_emb.shape[0] + item_emb.shape[0]
    D = user_emb.shape[1]
    E = edge_index.shape[1]
    NHALF = N // NC
    ROWS = NS * ZCH * ZR
    NPAD = NS * WPT

    RS = 50432
    src = edge_index[0]
    dst = edge_index[1]
    h0 = jnp.concatenate([user_emb, item_emb], axis=0)

    psrc, pdloc, counts, _ = _make_part_kernel(N, E, RS)(src, dst)

    deg_zeros = jnp.zeros((WPT,), jnp.float32)
    degs = _make_deg_kernel(E, NPAD)(edge_index.reshape(-1), deg_zeros)
    od = degs[:N, None]
    idg = degs[NPAD:NPAD + N, None]

    R = 2000
    grid = (N // R,)
    mat = pl.BlockSpec((R, D), lambda i: (i, 0))
    col = pl.BlockSpec((R, 1), lambda i: (i, 0))
    g0, inorm, ion = pl.pallas_call(
        _prep_body,
        grid=grid,
        in_specs=[mat, col, col],
        out_specs=[mat, col, col],
        out_shape=[
            jax.ShapeDtypeStruct((N, D), jnp.float32),
            jax.ShapeDtypeStruct((N, 1), jnp.float32),
            jax.ShapeDtypeStruct((N, 1), jnp.float32),
        ],
    )(h0, od, idg)

    layer = _make_layer_kernel(N, E, D, ROWS, RS)
    layer_zeros = jnp.zeros((128, D), jnp.float32)

    epi = pl.pallas_call(
        _epi_body,
        grid=grid,
        in_specs=[mat, col, col, mat],
        out_specs=[mat, mat],
        out_shape=[
            jax.ShapeDtypeStruct((N, D), jnp.float32),
            jax.ShapeDtypeStruct((N, D), jnp.float32),
        ],
    )
    fin = pl.pallas_call(
        _fin_body,
        grid=grid,
        in_specs=[mat, col, mat],
        out_specs=mat,
        out_shape=jax.ShapeDtypeStruct((N, D), jnp.float32),
    )

    g = g0
    s_acc = h0
    for k in range(3):
        aggp = layer(g, psrc, pdloc, counts, layer_zeros)
        agg = jnp.concatenate(
            [aggp[:NHALF], aggp[ROWS:ROWS + NHALF]], axis=0)
        if k < 2:
            g, s_acc = epi(agg, inorm, ion, s_acc)
        else:
            out = fin(agg, inorm, s_acc)

    return (out[: user_emb.shape[0]], out[user_emb.shape[0]:])


# speedup vs baseline: 8.1098x; 1.0703x over previous
import functools

import jax
import jax.numpy as jnp
from jax import lax
from jax.experimental import pallas as pl
from jax.experimental.pallas import tpu as pltpu, tpu_sc as plsc

NC = 2
NS = 16
C = 80
WPT = 3136
ZR = 112
ZCH = 14


def _make_deg_kernel(E, NPAD):
    EPT = E // NS
    n_chunks = EPT // C

    assert n_chunks % 2 == 1 and EPT % C == 0

    @functools.partial(
        pl.kernel,
        out_type=jax.ShapeDtypeStruct((NC * NPAD,), jnp.float32),
        mesh=plsc.VectorSubcoreMesh(core_axis_name="c", subcore_axis_name="s"),
        compiler_params=pltpu.CompilerParams(use_tc_tiling_on_sc=False, needs_layout_passes=False),
        scratch_types=[
            pltpu.VMEM((C,), jnp.int32),
            pltpu.VMEM((C,), jnp.int32),
            pltpu.VMEM((C,), jnp.float32),
            pltpu.VMEM((C,), jnp.float32),
            pltpu.VMEM((WPT,), jnp.float32),
            pltpu.VMEM_SHARED((NPAD,), jnp.float32),
            pltpu.SemaphoreType.DMA,
            pltpu.SemaphoreType.DMA,
        ],
    )
    def deg_kernel(edge_hbm, zeros_hbm, deg_hbm, idx_a, idx_b, ones_v,
                   zeros_v, stage_v, acc_sh, sem_i, sem_s):
        c = lax.axis_index("c")
        s = lax.axis_index("s")
        w0 = pl.multiple_of(s * WPT, 8)
        pltpu.sync_copy(zeros_hbm, stage_v)
        pltpu.sync_copy(stage_v, acc_sh.at[pl.ds(w0, WPT)])
        for j in range(C // 16):
            ones_v[pl.ds(16 * j, 16)] = jnp.ones((16,), jnp.float32)
            zeros_v[pl.ds(16 * j, 16)] = jnp.zeros((16,), jnp.float32)
            idx_b[pl.ds(16 * j, 16)] = jnp.zeros((16,), jnp.int32)
        plsc.subcore_barrier()

        e0 = c * E + s * EPT
        idxs = (idx_a, idx_b)
        pltpu.async_copy(zeros_v, acc_sh.at[idx_b], sem_s, add=True)
        pltpu.async_copy(edge_hbm.at[pl.ds(pl.multiple_of(e0, 8), C)], idx_a,
                         sem_i)

        def phase(e, t, n, issue_next):
            pltpu.make_async_copy(
                ones_v, acc_sh.at[idxs[n]], sem_s).wait()
            if issue_next:
                bn = pl.multiple_of(e0 + (e + 1) * C, 8)
                pltpu.async_copy(edge_hbm.at[pl.ds(bn, C)], idxs[n], sem_i)
            b = pl.multiple_of(e0 + e * C, 8)
            pltpu.make_async_copy(edge_hbm.at[pl.ds(b, C)], idxs[t],
                                  sem_i).wait()
            pltpu.async_copy(ones_v, acc_sh.at[idxs[t]], sem_s, add=True)

        def pair(p, carry):
            phase(2 * p, 0, 1, True)
            phase(2 * p + 1, 1, 0, True)
            return carry

        lax.fori_loop(0, (n_chunks - 1) // 2, pair, 0)
        phase(n_chunks - 1, 0, 1, False)
        pltpu.make_async_copy(ones_v, acc_sh.at[idx_a], sem_s).wait()
        plsc.subcore_barrier()
        o0 = pl.multiple_of(c * NPAD + w0, 8)
        pltpu.sync_copy(acc_sh.at[pl.ds(w0, WPT)], stage_v)
        pltpu.sync_copy(stage_v, deg_hbm.at[pl.ds(o0, WPT)])

    return deg_kernel


def _make_part_kernel(N, E, RS):
    NHALF = N // NC
    EPT = E // NS
    CL = 128
    n_full = EPT // CL
    TAIL = EPT - n_full * CL
    assert n_full % 2 == 0 and TAIL % 16 == 0 and 0 < TAIL <= CL

    HR = 512
    HPT = HR // NS

    @functools.partial(
        pl.kernel,
        out_type=[
            jax.ShapeDtypeStruct((NC * NS * RS,), jnp.int32),
            jax.ShapeDtypeStruct((NC * NS * RS,), jnp.int32),
            jax.ShapeDtypeStruct((NC * NS,), jnp.int32),
            jax.ShapeDtypeStruct((512,), jnp.int32),
            jax.ShapeDtypeStruct((NC * HR, 128), jnp.float32),
        ],
        mesh=plsc.VectorSubcoreMesh(core_axis_name="c", subcore_axis_name="s"),
        compiler_params=pltpu.CompilerParams(use_tc_tiling_on_sc=False, needs_layout_passes=False),
        scratch_types=[
            pltpu.VMEM((CL,), jnp.int32),
            pltpu.VMEM((CL,), jnp.int32),
            pltpu.VMEM((CL,), jnp.int32),
            pltpu.VMEM((CL,), jnp.int32),
            pltpu.VMEM((256,), jnp.int32),
            pltpu.VMEM((256,), jnp.int32),
            pltpu.VMEM((CL,), jnp.int32),
            pltpu.VMEM((CL,), jnp.int32),
            pltpu.VMEM((16,), jnp.int32),
            pltpu.VMEM((16,), jnp.int32),
            pltpu.VMEM((HR, 128), jnp.float32),
            pltpu.VMEM((128,), jnp.int32),
            pltpu.VMEM((HPT, 128), jnp.float32),
            pltpu.VMEM_SHARED((16,), jnp.int32),
            pltpu.VMEM_SHARED((HR, 128), jnp.float32),
            pltpu.SemaphoreType.DMA,
            pltpu.SemaphoreType.DMA,
            pltpu.SemaphoreType.DMA,
        ],
    )
    def part_kernel(src_hbm, dst_hbm, hzero_hbm, psrc_hbm, pdloc_hbm,
                    counts_hbm, waste_hbm, degs_hbm,
                    src_a, src_b, dst_a, dst_b, ring_s, ring_d,
                    dum_s, dum_d, cbuf, idxc, hist_v, hidx_v, hstage_v,
                    counts_sh, dacc_sh, sem_i, sem_f, sem_c):
        c = lax.axis_index("c")
        s = lax.axis_index("s")
        base_node = c * NHALF
        region = (c * NS + s) * RS
        e0 = s * EPT
        srcs = (src_a, src_b)
        dsts = (dst_a, dst_b)
        iota = lax.iota(jnp.int32, 16)
        fones = jnp.ones((16,), jnp.float32)
        for j in range(CL // 16):
            dum_s[pl.ds(16 * j, 16)] = jnp.zeros((16,), jnp.int32)
            dum_d[pl.ds(16 * j, 16)] = jnp.full((16,), NHALF, jnp.int32)
        pltpu.sync_copy(hzero_hbm, hist_v)
        pltpu.sync_copy(hzero_hbm.at[pl.ds(0, HPT)], hstage_v)
        pltpu.sync_copy(hstage_v, dacc_sh.at[pl.ds(s * HPT, HPT)])
        plsc.subcore_barrier()

        def flush_from(ref_s, ref_d, off, w):
            pltpu.make_async_copy(ring_s.at[pl.ds(0, CL)],
                                  waste_hbm.at[pl.ds(0, CL)], sem_f).wait()
            pltpu.make_async_copy(ring_d.at[pl.ds(0, CL)],
                                  waste_hbm.at[pl.ds(0, CL)], sem_f).wait()
            b = pl.multiple_of(region + w * CL, 8)
            pltpu.async_copy(ref_s.at[pl.ds(off, CL)],
                             psrc_hbm.at[pl.ds(b, CL)], sem_f)
            pltpu.async_copy(ref_d.at[pl.ds(off, CL)],
                             pdloc_hbm.at[pl.ds(b, CL)], sem_f)

        pltpu.async_copy(ring_s.at[pl.ds(0, CL)],
                         waste_hbm.at[pl.ds(0, CL)], sem_f)
        pltpu.async_copy(ring_d.at[pl.ds(0, CL)],
                         waste_hbm.at[pl.ds(128, CL)], sem_f)
        pltpu.async_copy(ring_s.at[pl.ds(128, CL)],
                         waste_hbm.at[pl.ds(256, CL)], sem_f)
        pltpu.async_copy(ring_d.at[pl.ds(128, CL)],
                         waste_hbm.at[pl.ds(384, CL)], sem_f)

        def group(sv, dv, cnt):
            ei = jnp.where(c == 0, sv, dv)
            plsc.addupdate_scatter(hist_v, [ei >> 7, ei & 127], fones)
            dl = dv - base_node
            flag = (dl >= 0) & (dl < NHALF)
            fi = jnp.where(flag, 1, 0)
            pos = (cnt + plsc.cumsum(fi) - 1) & 255
            plsc.store_scatter(ring_s, [pos], sv, mask=flag)
            plsc.store_scatter(ring_d, [pos], dl, mask=flag)
            return cnt + jnp.sum(fi)

        def consume(t, n16, cnt0):
            w0 = cnt0 >> 7
            cnt = cnt0
            for j in range(n16):
                sv = srcs[t][pl.ds(16 * j, 16)]
                dv = dsts[t][pl.ds(16 * j, 16)]
                cnt = group(sv, dv, cnt)
            w1 = cnt >> 7

            @pl.when(w1 > w0)
            def _():
                flush_from(ring_s, ring_d,
                           pl.multiple_of((w0 & 1) * CL, 8), w0)

            return cnt

        def phase(e, t, n, issue_next, cnt):
            if issue_next:
                bn = pl.multiple_of(e0 + (e + 1) * CL, 8)
                pltpu.async_copy(src_hbm.at[pl.ds(bn, CL)], srcs[n], sem_i)
                pltpu.async_copy(dst_hbm.at[pl.ds(bn, CL)], dsts[n], sem_i)
            b = pl.multiple_of(e0 + e * CL, 8)
            pltpu.make_async_copy(src_hbm.at[pl.ds(b, CL)], srcs[t],
                                  sem_i).wait()
            pltpu.make_async_copy(dst_hbm.at[pl.ds(b, CL)], dsts[t],
                                  sem_i).wait()
            return consume(t, CL // 16, cnt)

        b0 = pl.multiple_of(e0, 8)
        pltpu.async_copy(src_hbm.at[pl.ds(b0, CL)], src_a, sem_i)
        pltpu.async_copy(dst_hbm.at[pl.ds(b0, CL)], dst_a, sem_i)

        def pair(p, cnt):
            cnt = phase(2 * p, 0, 1, True, cnt)
            cnt = phase(2 * p + 1, 1, 0, True, cnt)
            return cnt

        cnt = lax.fori_loop(0, (n_full - 2) // 2, pair, 0)
        cnt = phase(n_full - 2, 0, 1, True, cnt)
        cnt = phase(n_full - 1, 1, 0, False, cnt)
        bt = pl.multiple_of(e0 + n_full * CL, 8)
        pltpu.sync_copy(src_hbm.at[pl.ds(bt, TAIL)], src_a.at[pl.ds(0, TAIL)])
        pltpu.sync_copy(dst_hbm.at[pl.ds(bt, TAIL)], dst_a.at[pl.ds(0, TAIL)])
        cnt = consume(0, TAIL // 16, cnt)

        w = cnt >> 7
        rem = cnt & 127
        hb = pl.multiple_of((w & 1) * CL, 8)
        for j in range(CL // 16):
            lane = iota + 16 * j
            over = lane >= rem
            gs = ring_s[pl.ds(hb + 16 * j, 16)]
            gd = ring_d[pl.ds(hb + 16 * j, 16)]
            ring_s[pl.ds(hb + 16 * j, 16)] = jnp.where(over, 0, gs)
            ring_d[pl.ds(hb + 16 * j, 16)] = jnp.where(over, NHALF, gd)
        flush_from(ring_s, ring_d, hb, w)
        nw = w + 1

        @pl.when(nw % 2 == 1)
        def _():
            flush_from(dum_s, dum_d, 0, nw)

        nw2 = nw + (nw % 2)
        for _ in range(4):
            pltpu.make_async_copy(ring_s.at[pl.ds(0, CL)],
                                  waste_hbm.at[pl.ds(0, CL)], sem_f).wait()
        for k in range(HR // 128):
            for j in range(8):
                hidx_v[pl.ds(16 * j, 16)] = iota + (16 * j + 128 * k)
            pltpu.sync_copy(hist_v.at[pl.ds(128 * k, 128)],
                            dacc_sh.at[hidx_v], add=True)
        cbuf[pl.ds(0, 16)] = jnp.full((16,), nw2 * CL, jnp.int32)
        idxc[pl.ds(0, 16)] = jnp.full((16,), s, jnp.int32)
        pltpu.async_copy(cbuf, counts_sh.at[idxc], sem_c).wait()
        plsc.subcore_barrier()

        @pl.when(s == 0)
        def _():
            pltpu.sync_copy(counts_sh, counts_hbm.at[pl.ds(c * NS, NS)])

        pltpu.sync_copy(dacc_sh.at[pl.ds(s * HPT, HPT)], hstage_v)
        pltpu.sync_copy(hstage_v,
                        degs_hbm.at[pl.ds(c * HR + s * HPT, HPT)])

    return part_kernel


def _make_layer_kernel(N, E, D, ROWS, RS):
    NHALF = N // NC
    CL = 128
    RPT = ROWS // NS

    @functools.partial(
        pl.kernel,
        out_type=jax.ShapeDtypeStruct((NC * ROWS, D), jnp.float32),
        mesh=plsc.VectorSubcoreMesh(core_axis_name="c", subcore_axis_name="s"),
        compiler_params=pltpu.CompilerParams(use_tc_tiling_on_sc=False, needs_layout_passes=False),
        scratch_types=[
            pltpu.VMEM((CL,), jnp.int32),
            pltpu.VMEM((CL,), jnp.int32),
            pltpu.VMEM((CL,), jnp.int32),
            pltpu.VMEM((CL,), jnp.int32),
            pltpu.VMEM((CL, D), jnp.float32),
            pltpu.VMEM((CL, D), jnp.float32),
            pltpu.VMEM((16,), jnp.int32),
            pltpu.VMEM((ZR, D), jnp.float32),
            pltpu.VMEM_SHARED((ROWS, D), jnp.float32),
            pltpu.SemaphoreType.DMA,
            pltpu.SemaphoreType.DMA,
            pltpu.SemaphoreType.DMA,
        ],
    )
    def layer_kernel(g_hbm, psrc_hbm, pdloc_hbm, counts_hbm, zeros_hbm,
                     out_hbm, src_a, src_b, dloc_a, dloc_b,
                     rows_a, rows_b, cnt_v, stage_v, acc_sh,
                     sem_i, sem_g, sem_s):
        c = lax.axis_index("c")
        s = lax.axis_index("s")
        r0 = s * RPT
        region = (c * NS + s) * RS
        pltpu.sync_copy(zeros_hbm.at[pl.ds(0, ZR)], stage_v)
        for k in range(ZCH):
            pltpu.sync_copy(stage_v, acc_sh.at[pl.ds(r0 + k * ZR, ZR)])
        plsc.subcore_barrier()

        pltpu.sync_copy(
            counts_hbm.at[pl.ds(pl.multiple_of(c * NS, 8), NS)], cnt_v)
        cv = cnt_v[pl.ds(0, 16)]
        cw = jnp.sum(jnp.where(lax.iota(jnp.int32, 16) == s, cv, 0))
        n_t = cw >> 7

        srcs = (src_a, src_b)
        dlocs = (dloc_a, dloc_b)
        rows = (rows_a, rows_b)

        for j in range(CL // 16):
            dloc_b[pl.ds(16 * j, 16)] = jnp.zeros((16,), jnp.int32)
        pltpu.sync_copy(zeros_hbm, rows_b)
        pltpu.async_copy(rows_b, acc_sh.at[dloc_b], sem_s, add=True)
        b0 = pl.multiple_of(region, 8)
        pltpu.sync_copy(psrc_hbm.at[pl.ds(b0, CL)], src_a)
        pltpu.sync_copy(pdloc_hbm.at[pl.ds(b0, CL)], dloc_a)
        pltpu.async_copy(g_hbm.at[src_a], rows_a, sem_g)

        def phase(e, t, n, issue_next):
            pltpu.make_async_copy(
                rows[n], acc_sh.at[dlocs[n]], sem_s).wait()
            if issue_next:
                bn = pl.multiple_of(region + (e + 1) * CL, 8)
                pltpu.async_copy(psrc_hbm.at[pl.ds(bn, CL)], srcs[n], sem_i)
                pltpu.async_copy(pdloc_hbm.at[pl.ds(bn, CL)], dlocs[n], sem_i)
            pltpu.make_async_copy(g_hbm.at[srcs[t]], rows[t], sem_g).wait()
            if issue_next:
                bn = pl.multiple_of(region + (e + 1) * CL, 8)
                pltpu.make_async_copy(psrc_hbm.at[pl.ds(bn, CL)], srcs[n],
                                      sem_i).wait()
                pltpu.make_async_copy(pdloc_hbm.at[pl.ds(bn, CL)], dlocs[n],
                                      sem_i).wait()
                pltpu.async_copy(g_hbm.at[srcs[n]], rows[n], sem_g)
            pltpu.async_copy(rows[t], acc_sh.at[dlocs[t]], sem_s, add=True)

        def pair(p, carry):
            phase(2 * p, 0, 1, True)
            phase(2 * p + 1, 1, 0, True)
            return carry

        lax.fori_loop(0, (n_t - 2) // 2, pair, 0)
        phase(n_t - 2, 0, 1, True)
        phase(n_t - 1, 1, 0, False)
        pltpu.make_async_copy(rows_b, acc_sh.at[dloc_b], sem_s).wait()
        plsc.subcore_barrier()
        o0 = c * ROWS + r0
        for k in range(ZCH):
            pltpu.sync_copy(acc_sh.at[pl.ds(r0 + k * ZR, ZR)], stage_v)
            pltpu.sync_copy(
                stage_v, out_hbm.at[pl.ds(pl.multiple_of(o0 + k * ZR, 8), ZR)])

    return layer_kernel


def _prep_body(h_ref, od_ref, id_ref, g0_ref, inorm_ref, ion_ref):
    on = lax.rsqrt(jnp.maximum(od_ref[...], 1.0))
    inn = lax.rsqrt(jnp.maximum(id_ref[...], 1.0))
    g0_ref[...] = h_ref[...] * on
    inorm_ref[...] = inn
    ion_ref[...] = inn * on


def _epi_body(agg_ref, inorm_ref, ion_ref, s_ref, g_ref, snew_ref):
    a = agg_ref[...]
    g_ref[...] = a * ion_ref[...]
    snew_ref[...] = s_ref[...] + a * inorm_ref[...]


def _fin_body(agg_ref, inorm_ref, s_ref, o_ref):
    o_ref[...] = (s_ref[...] + agg_ref[...] * inorm_ref[...]) * 0.25


def kernel(user_emb, item_emb, edge_index):
    N = user_emb.shape[0] + item_emb.shape[0]
    D = user_emb.shape[1]
    E = edge_index.shape[1]
    NHALF = N // NC
    ROWS = NS * ZCH * ZR
    NPAD = NS * WPT

    RS = 50432
    src = edge_index[0]
    dst = edge_index[1]
    h0 = jnp.concatenate([user_emb, item_emb], axis=0)

    hzero = jnp.zeros((512, 128), jnp.float32)
    psrc, pdloc, counts, _, degs2 = _make_part_kernel(N, E, RS)(
        src, dst, hzero)

    degs = degs2.reshape(NC, 512 * 128)
    od = degs[0, :N, None]
    idg = degs[1, :N, None]

    R = 2000
    grid = (N // R,)
    mat = pl.BlockSpec((R, D), lambda i: (i, 0))
    col = pl.BlockSpec((R, 1), lambda i: (i, 0))
    g0, inorm, ion = pl.pallas_call(
        _prep_body,
        grid=grid,
        in_specs=[mat, col, col],
        out_specs=[mat, col, col],
        out_shape=[
            jax.ShapeDtypeStruct((N, D), jnp.float32),
            jax.ShapeDtypeStruct((N, 1), jnp.float32),
            jax.ShapeDtypeStruct((N, 1), jnp.float32),
        ],
    )(h0, od, idg)

    layer = _make_layer_kernel(N, E, D, ROWS, RS)
    layer_zeros = jnp.zeros((128, D), jnp.float32)

    epi = pl.pallas_call(
        _epi_body,
        grid=grid,
        in_specs=[mat, col, col, mat],
        out_specs=[mat, mat],
        out_shape=[
            jax.ShapeDtypeStruct((N, D), jnp.float32),
            jax.ShapeDtypeStruct((N, D), jnp.float32),
        ],
    )
    fin = pl.pallas_call(
        _fin_body,
        grid=grid,
        in_specs=[mat, col, mat],
        out_specs=mat,
        out_shape=jax.ShapeDtypeStruct((N, D), jnp.float32),
    )

    g = g0
    s_acc = h0
    for k in range(3):
        aggp = layer(g, psrc, pdloc, counts, layer_zeros)
        agg = jnp.concatenate(
            [aggp[:NHALF], aggp[ROWS:ROWS + NHALF]], axis=0)
        if k < 2:
            g, s_acc = epi(agg, inorm, ion, s_acc)
        else:
            out = fin(agg, inorm, s_acc)

    return (out[: user_emb.shape[0]], out[user_emb.shape[0]:])


# speedup vs baseline: 8.1104x; 1.0001x over previous
import functools

import jax
import jax.numpy as jnp
from jax import lax
from jax.experimental import pallas as pl
from jax.experimental.pallas import tpu as pltpu, tpu_sc as plsc

NC = 2
NS = 16
ZR = 112
ZCH = 14


def _make_part_kernel(N, E, RS):
    NHALF = N // NC
    EPT = E // NS
    CL = 128
    n_full = EPT // CL
    TAIL = EPT - n_full * CL
    assert n_full % 2 == 0 and TAIL % 16 == 0 and 0 < TAIL <= CL

    HR = 512
    HPT = HR // NS

    @functools.partial(
        pl.kernel,
        out_type=[
            jax.ShapeDtypeStruct((NC * NS * RS,), jnp.int32),
            jax.ShapeDtypeStruct((NC * NS * RS,), jnp.int32),
            jax.ShapeDtypeStruct((NC * NS,), jnp.int32),
            jax.ShapeDtypeStruct((512,), jnp.int32),
            jax.ShapeDtypeStruct((NC * HR, 128), jnp.float32),
        ],
        mesh=plsc.VectorSubcoreMesh(core_axis_name="c", subcore_axis_name="s"),
        compiler_params=pltpu.CompilerParams(use_tc_tiling_on_sc=False, needs_layout_passes=False),
        scratch_types=[
            pltpu.VMEM((CL,), jnp.int32),
            pltpu.VMEM((CL,), jnp.int32),
            pltpu.VMEM((CL,), jnp.int32),
            pltpu.VMEM((CL,), jnp.int32),
            pltpu.VMEM((256,), jnp.int32),
            pltpu.VMEM((256,), jnp.int32),
            pltpu.VMEM((CL,), jnp.int32),
            pltpu.VMEM((CL,), jnp.int32),
            pltpu.VMEM((16,), jnp.int32),
            pltpu.VMEM((16,), jnp.int32),
            pltpu.VMEM((HR, 128), jnp.float32),
            pltpu.VMEM((128,), jnp.int32),
            pltpu.VMEM((HPT, 128), jnp.float32),
            pltpu.VMEM_SHARED((16,), jnp.int32),
            pltpu.VMEM_SHARED((HR, 128), jnp.float32),
            pltpu.SemaphoreType.DMA,
            pltpu.SemaphoreType.DMA,
            pltpu.SemaphoreType.DMA,
        ],
    )
    def part_kernel(src_hbm, dst_hbm, hzero_hbm, psrc_hbm, pdloc_hbm,
                    counts_hbm, waste_hbm, degs_hbm,
                    src_a, src_b, dst_a, dst_b, ring_s, ring_d,
                    dum_s, dum_d, cbuf, idxc, hist_v, hidx_v, hstage_v,
                    counts_sh, dacc_sh, sem_i, sem_f, sem_c):
        c = lax.axis_index("c")
        s = lax.axis_index("s")
        base_node = c * NHALF
        region = (c * NS + s) * RS
        e0 = s * EPT
        srcs = (src_a, src_b)
        dsts = (dst_a, dst_b)
        iota = lax.iota(jnp.int32, 16)
        fones = jnp.ones((16,), jnp.float32)
        for j in range(CL // 16):
            dum_s[pl.ds(16 * j, 16)] = jnp.zeros((16,), jnp.int32)
            dum_d[pl.ds(16 * j, 16)] = jnp.full((16,), NHALF, jnp.int32)
        pltpu.sync_copy(hzero_hbm, hist_v)
        pltpu.sync_copy(hzero_hbm.at[pl.ds(0, HPT)], hstage_v)
        pltpu.sync_copy(hstage_v, dacc_sh.at[pl.ds(s * HPT, HPT)])
        plsc.subcore_barrier()

        def flush_from(ref_s, ref_d, off, w):
            pltpu.make_async_copy(ring_s.at[pl.ds(0, CL)],
                                  waste_hbm.at[pl.ds(0, CL)], sem_f).wait()
            pltpu.make_async_copy(ring_d.at[pl.ds(0, CL)],
                                  waste_hbm.at[pl.ds(0, CL)], sem_f).wait()
            b = pl.multiple_of(region + w * CL, 8)
            pltpu.async_copy(ref_s.at[pl.ds(off, CL)],
                             psrc_hbm.at[pl.ds(b, CL)], sem_f)
            pltpu.async_copy(ref_d.at[pl.ds(off, CL)],
                             pdloc_hbm.at[pl.ds(b, CL)], sem_f)

        pltpu.async_copy(ring_s.at[pl.ds(0, CL)],
                         waste_hbm.at[pl.ds(0, CL)], sem_f)
        pltpu.async_copy(ring_d.at[pl.ds(0, CL)],
                         waste_hbm.at[pl.ds(128, CL)], sem_f)
        pltpu.async_copy(ring_s.at[pl.ds(128, CL)],
                         waste_hbm.at[pl.ds(256, CL)], sem_f)
        pltpu.async_copy(ring_d.at[pl.ds(128, CL)],
                         waste_hbm.at[pl.ds(384, CL)], sem_f)

        def group(sv, dv, cnt):
            ei = jnp.where(c == 0, sv, dv)
            plsc.addupdate_scatter(hist_v, [ei >> 7, ei & 127], fones)
            dl = dv - base_node
            flag = (dl >= 0) & (dl < NHALF)
            fi = jnp.where(flag, 1, 0)
            pos = (cnt + plsc.cumsum(fi) - 1) & 255
            plsc.store_scatter(ring_s, [pos], sv, mask=flag)
            plsc.store_scatter(ring_d, [pos], dl, mask=flag)
            return cnt + jnp.sum(fi)

        def consume(t, n16, cnt0):
            w0 = cnt0 >> 7
            cnt = cnt0
            for j in range(n16):
                sv = srcs[t][pl.ds(16 * j, 16)]
                dv = dsts[t][pl.ds(16 * j, 16)]
                cnt = group(sv, dv, cnt)
            w1 = cnt >> 7

            @pl.when(w1 > w0)
            def _():
                flush_from(ring_s, ring_d,
                           pl.multiple_of((w0 & 1) * CL, 8), w0)

            return cnt

        def phase(e, t, n, issue_next, cnt):
            if issue_next:
                bn = pl.multiple_of(e0 + (e + 1) * CL, 8)
                pltpu.async_copy(src_hbm.at[pl.ds(bn, CL)], srcs[n], sem_i)
                pltpu.async_copy(dst_hbm.at[pl.ds(bn, CL)], dsts[n], sem_i)
            b = pl.multiple_of(e0 + e * CL, 8)
            pltpu.make_async_copy(src_hbm.at[pl.ds(b, CL)], srcs[t],
                                  sem_i).wait()
            pltpu.make_async_copy(dst_hbm.at[pl.ds(b, CL)], dsts[t],
                                  sem_i).wait()
            return consume(t, CL // 16, cnt)

        b0 = pl.multiple_of(e0, 8)
        pltpu.async_copy(src_hbm.at[pl.ds(b0, CL)], src_a, sem_i)
        pltpu.async_copy(dst_hbm.at[pl.ds(b0, CL)], dst_a, sem_i)

        def pair(p, cnt):
            cnt = phase(2 * p, 0, 1, True, cnt)
            cnt = phase(2 * p + 1, 1, 0, True, cnt)
            return cnt

        cnt = lax.fori_loop(0, (n_full - 2) // 2, pair, 0)
        cnt = phase(n_full - 2, 0, 1, True, cnt)
        cnt = phase(n_full - 1, 1, 0, False, cnt)
        bt = pl.multiple_of(e0 + n_full * CL, 8)
        pltpu.sync_copy(src_hbm.at[pl.ds(bt, TAIL)], src_a.at[pl.ds(0, TAIL)])
        pltpu.sync_copy(dst_hbm.at[pl.ds(bt, TAIL)], dst_a.at[pl.ds(0, TAIL)])
        cnt = consume(0, TAIL // 16, cnt)

        w = cnt >> 7
        rem = cnt & 127
        hb = pl.multiple_of((w & 1) * CL, 8)
        for j in range(CL // 16):
            lane = iota + 16 * j
            over = lane >= rem
            gs = ring_s[pl.ds(hb + 16 * j, 16)]
            gd = ring_d[pl.ds(hb + 16 * j, 16)]
            ring_s[pl.ds(hb + 16 * j, 16)] = jnp.where(over, 0, gs)
            ring_d[pl.ds(hb + 16 * j, 16)] = jnp.where(over, NHALF, gd)
        flush_from(ring_s, ring_d, hb, w)
        nw = w + 1

        @pl.when(nw % 2 == 1)
        def _():
            flush_from(dum_s, dum_d, 0, nw)

        nw2 = nw + (nw % 2)
        for _ in range(4):
            pltpu.make_async_copy(ring_s.at[pl.ds(0, CL)],
                                  waste_hbm.at[pl.ds(0, CL)], sem_f).wait()
        for k in range(HR // 128):
            for j in range(8):
                hidx_v[pl.ds(16 * j, 16)] = iota + (16 * j + 128 * k)
            pltpu.sync_copy(hist_v.at[pl.ds(128 * k, 128)],
                            dacc_sh.at[hidx_v], add=True)
        cbuf[pl.ds(0, 16)] = jnp.full((16,), nw2 * CL, jnp.int32)
        idxc[pl.ds(0, 16)] = jnp.full((16,), s, jnp.int32)
        pltpu.async_copy(cbuf, counts_sh.at[idxc], sem_c).wait()
        plsc.subcore_barrier()

        @pl.when(s == 0)
        def _():
            pltpu.sync_copy(counts_sh, counts_hbm.at[pl.ds(c * NS, NS)])

        pltpu.sync_copy(dacc_sh.at[pl.ds(s * HPT, HPT)], hstage_v)
        pltpu.sync_copy(hstage_v,
                        degs_hbm.at[pl.ds(c * HR + s * HPT, HPT)])

    return part_kernel


def _make_layer_kernel(N, E, D, ROWS, RS):
    NHALF = N // NC
    CL = 128
    RPT = ROWS // NS

    @functools.partial(
        pl.kernel,
        out_type=jax.ShapeDtypeStruct((NC * ROWS, D), jnp.float32),
        mesh=plsc.VectorSubcoreMesh(core_axis_name="c", subcore_axis_name="s"),
        compiler_params=pltpu.CompilerParams(use_tc_tiling_on_sc=False, needs_layout_passes=False),
        scratch_types=[
            pltpu.VMEM((CL,), jnp.int32),
            pltpu.VMEM((CL,), jnp.int32),
            pltpu.VMEM((CL,), jnp.int32),
            pltpu.VMEM((CL,), jnp.int32),
            pltpu.VMEM((CL, D), jnp.float32),
            pltpu.VMEM((CL, D), jnp.float32),
            pltpu.VMEM((16,), jnp.int32),
            pltpu.VMEM((ZR, D), jnp.float32),
            pltpu.VMEM_SHARED((ROWS, D), jnp.float32),
            pltpu.SemaphoreType.DMA,
            pltpu.SemaphoreType.DMA,
            pltpu.SemaphoreType.DMA,
        ],
    )
    def layer_kernel(g_hbm, psrc_hbm, pdloc_hbm, counts_hbm, zeros_hbm,
                     out_hbm, src_a, src_b, dloc_a, dloc_b,
                     rows_a, rows_b, cnt_v, stage_v, acc_sh,
                     sem_i, sem_g, sem_s):
        c = lax.axis_index("c")
        s = lax.axis_index("s")
        r0 = s * RPT
        region = (c * NS + s) * RS
        pltpu.sync_copy(zeros_hbm.at[pl.ds(0, ZR)], stage_v)
        for k in range(ZCH):
            pltpu.sync_copy(stage_v, acc_sh.at[pl.ds(r0 + k * ZR, ZR)])
        plsc.subcore_barrier()

        pltpu.sync_copy(
            counts_hbm.at[pl.ds(pl.multiple_of(c * NS, 8), NS)], cnt_v)
        cv = cnt_v[pl.ds(0, 16)]
        cw = jnp.sum(jnp.where(lax.iota(jnp.int32, 16) == s, cv, 0))
        n_t = cw >> 7

        srcs = (src_a, src_b)
        dlocs = (dloc_a, dloc_b)
        rows = (rows_a, rows_b)

        for j in range(CL // 16):
            dloc_b[pl.ds(16 * j, 16)] = jnp.zeros((16,), jnp.int32)
        pltpu.sync_copy(zeros_hbm, rows_b)
        pltpu.async_copy(rows_b, acc_sh.at[dloc_b], sem_s, add=True)
        b0 = pl.multiple_of(region, 8)
        pltpu.sync_copy(psrc_hbm.at[pl.ds(b0, CL)], src_a)
        pltpu.sync_copy(pdloc_hbm.at[pl.ds(b0, CL)], dloc_a)
        pltpu.async_copy(g_hbm.at[src_a], rows_a, sem_g)

        def phase(e, t, n, issue_next):
            pltpu.make_async_copy(
                rows[n], acc_sh.at[dlocs[n]], sem_s).wait()
            if issue_next:
                bn = pl.multiple_of(region + (e + 1) * CL, 8)
                pltpu.async_copy(psrc_hbm.at[pl.ds(bn, CL)], srcs[n], sem_i)
                pltpu.async_copy(pdloc_hbm.at[pl.ds(bn, CL)], dlocs[n], sem_i)
            pltpu.make_async_copy(g_hbm.at[srcs[t]], rows[t], sem_g).wait()
            if issue_next:
                bn = pl.multiple_of(region + (e + 1) * CL, 8)
                pltpu.make_async_copy(psrc_hbm.at[pl.ds(bn, CL)], srcs[n],
                                      sem_i).wait()
                pltpu.make_async_copy(pdloc_hbm.at[pl.ds(bn, CL)], dlocs[n],
                                      sem_i).wait()
                pltpu.async_copy(g_hbm.at[srcs[n]], rows[n], sem_g)
            pltpu.async_copy(rows[t], acc_sh.at[dlocs[t]], sem_s, add=True)

        def pair(p, carry):
            phase(2 * p, 0, 1, True)
            phase(2 * p + 1, 1, 0, True)
            return carry

        lax.fori_loop(0, (n_t - 2) // 2, pair, 0)
        phase(n_t - 2, 0, 1, True)
        phase(n_t - 1, 1, 0, False)
        pltpu.make_async_copy(rows_b, acc_sh.at[dloc_b], sem_s).wait()
        plsc.subcore_barrier()
        o0 = c * ROWS + r0
        for k in range(ZCH):
            pltpu.sync_copy(acc_sh.at[pl.ds(r0 + k * ZR, ZR)], stage_v)
            pltpu.sync_copy(
                stage_v, out_hbm.at[pl.ds(pl.multiple_of(o0 + k * ZR, 8), ZR)])

    return layer_kernel


def _prep_body(h_ref, od_ref, id_ref, g0_ref, inorm_ref, ion_ref):
    on = lax.rsqrt(jnp.maximum(od_ref[...], 1.0))
    inn = lax.rsqrt(jnp.maximum(id_ref[...], 1.0))
    g0_ref[...] = h_ref[...] * on
    inorm_ref[...] = inn
    ion_ref[...] = inn * on


def _epi_body(agg_ref, inorm_ref, ion_ref, s_ref, g_ref, snew_ref):
    a = agg_ref[...]
    g_ref[...] = a * ion_ref[...]
    snew_ref[...] = s_ref[...] + a * inorm_ref[...]


def _fin_body(agg_ref, inorm_ref, s_ref, o_ref):
    o_ref[...] = (s_ref[...] + agg_ref[...] * inorm_ref[...]) * 0.25


def kernel(user_emb, item_emb, edge_index):
    N = user_emb.shape[0] + item_emb.shape[0]
    D = user_emb.shape[1]
    E = edge_index.shape[1]
    NHALF = N // NC
    ROWS = NS * ZCH * ZR

    RS = 50432
    src = edge_index[0]
    dst = edge_index[1]
    h0 = jnp.concatenate([user_emb, item_emb], axis=0)

    hzero = jnp.zeros((512, 128), jnp.float32)
    psrc, pdloc, counts, _, degs2 = _make_part_kernel(N, E, RS)(
        src, dst, hzero)

    degs = degs2.reshape(NC, 512 * 128)
    od = degs[0, :N, None]
    idg = degs[1, :N, None]

    R = 2000
    grid = (N // R,)
    mat = pl.BlockSpec((R, D), lambda i: (i, 0))
    col = pl.BlockSpec((R, 1), lambda i: (i, 0))
    g0, inorm, ion = pl.pallas_call(
        _prep_body,
        grid=grid,
        in_specs=[mat, col, col],
        out_specs=[mat, col, col],
        out_shape=[
            jax.ShapeDtypeStruct((N, D), jnp.float32),
            jax.ShapeDtypeStruct((N, 1), jnp.float32),
            jax.ShapeDtypeStruct((N, 1), jnp.float32),
        ],
    )(h0, od, idg)

    layer = _make_layer_kernel(N, E, D, ROWS, RS)
    layer_zeros = jnp.zeros((128, D), jnp.float32)

    epi = pl.pallas_call(
        _epi_body,
        grid=grid,
        in_specs=[mat, col, col, mat],
        out_specs=[mat, mat],
        out_shape=[
            jax.ShapeDtypeStruct((N, D), jnp.float32),
            jax.ShapeDtypeStruct((N, D), jnp.float32),
        ],
    )
    fin = pl.pallas_call(
        _fin_body,
        grid=grid,
        in_specs=[mat, col, mat],
        out_specs=mat,
        out_shape=jax.ShapeDtypeStruct((N, D), jnp.float32),
    )

    g = g0
    s_acc = h0
    for k in range(3):
        aggp = layer(g, psrc, pdloc, counts, layer_zeros)
        agg = jnp.concatenate(
            [aggp[:NHALF], aggp[ROWS:ROWS + NHALF]], axis=0)
        if k < 2:
            g, s_acc = epi(agg, inorm, ion, s_acc)
        else:
            out = fin(agg, inorm, s_acc)

    return (out[: user_emb.shape[0]], out[user_emb.shape[0]:])
